# baseline (speedup 1.0000x reference)
import functools

import jax
import jax.numpy as jnp
from jax import lax
from jax.experimental import pallas as pl
from jax.experimental.pallas import tpu as pltpu
from jax.experimental.pallas import tpu_sc as plsc

D_MODEL = 64
SCALE = 8.0
G = 128
NC = 2
NS = 16
NW = NC * NS


@functools.partial(jax.jit, static_argnames=())
def _gather_scale(idx2d, table):
    n_groups = idx2d.shape[0]
    gpw = n_groups // NW
    mesh = plsc.VectorSubcoreMesh(core_axis_name="c", subcore_axis_name="s")

    @functools.partial(
        pl.kernel,
        mesh=mesh,
        out_type=jax.ShapeDtypeStruct((n_groups * G, D_MODEL), jnp.float32),
        scratch_types=[
            pltpu.VMEM((gpw, G), jnp.int32),
            pltpu.VMEM((G, D_MODEL), jnp.float32),
            pltpu.SemaphoreType.DMA,
        ],
        compiler_params=pltpu.CompilerParams(use_tc_tiling_on_sc=False),
    )
    def k(idx_hbm, table_hbm, out_hbm, idx_v, rows_v, sem):
        wid = lax.axis_index("s") * NC + lax.axis_index("c")
        g0 = wid * gpw
        pltpu.sync_copy(idx_hbm.at[pl.ds(g0, gpw)], idx_v)

        def group_body(g, carry):
            pltpu.async_copy(table_hbm.at[idx_v.at[g]], rows_v, sem).wait()

            def scale_row(r, c2):
                for j in range(D_MODEL // 16):
                    sl = pl.ds(j * 16, 16)
                    rows_v[r, sl] = rows_v[r, sl] * SCALE
                return c2

            lax.fori_loop(0, G, scale_row, 0, unroll=4)
            pltpu.sync_copy(rows_v, out_hbm.at[pl.ds((g0 + g) * G, G)])
            return carry

        lax.fori_loop(0, gpw, group_body, 0)

    return k(idx2d, table)


def kernel(x, emb_weight):
    b, s = x.shape
    idx2d = x.reshape(b * s // G, G).astype(jnp.int32)
    out = _gather_scale(idx2d, emb_weight)
    return out.reshape(b, s, D_MODEL)


# speedup vs baseline: 1.1641x; 1.1641x over previous
import functools

import jax
import jax.numpy as jnp
from jax import lax
from jax.experimental import pallas as pl
from jax.experimental.pallas import tpu as pltpu
from jax.experimental.pallas import tpu_sc as plsc

D_MODEL = 64
SCALE = 8.0
G = 128
NC = 2
NS = 16
NW = NC * NS
NBUF = 8
DEPTH = 4


def _gather_scale(idx2d, table):
    n_groups = idx2d.shape[0]
    gpw = n_groups // NW
    assert gpw % NBUF == 0
    mesh = plsc.VectorSubcoreMesh(core_axis_name="c", subcore_axis_name="s")

    @functools.partial(
        pl.kernel,
        mesh=mesh,
        out_type=jax.ShapeDtypeStruct((n_groups * G, D_MODEL), jnp.float32),
        scratch_types=(
            [pltpu.VMEM((gpw, G), jnp.int32)]
            + [pltpu.VMEM((G, D_MODEL), jnp.float32) for _ in range(NBUF)]
            + [pltpu.SemaphoreType.DMA for _ in range(2 * NBUF)]
        ),
        compiler_params=pltpu.CompilerParams(use_tc_tiling_on_sc=False),
    )
    def k(idx_hbm, table_hbm, out_hbm, idx_v, *rest):
        bufs = rest[:NBUF]
        gsem = rest[NBUF:2 * NBUF]
        ssem = rest[2 * NBUF:]
        wid = lax.axis_index("s") * NC + lax.axis_index("c")
        g0 = wid * gpw
        pltpu.sync_copy(idx_hbm.at[pl.ds(g0, gpw)], idx_v)

        def gather(g, b):
            pltpu.async_copy(table_hbm.at[idx_v.at[g]], bufs[b], gsem[b])

        def wait_gather(g, b):
            pltpu.make_async_copy(
                table_hbm.at[idx_v.at[g]], bufs[b], gsem[b]).wait()

        def store(g, b):
            pltpu.async_copy(
                bufs[b], out_hbm.at[pl.ds((g0 + g) * G, G)], ssem[b])

        def wait_store(g, b):
            pltpu.make_async_copy(
                bufs[b], out_hbm.at[pl.ds((g0 + g) * G, G)], ssem[b]).wait()

        for b in range(DEPTH):
            gather(b, b)

        def block(kb, carry):
            for b in range(NBUF):
                g = kb * NBUF + b
                wait_gather(g, b)

                def scale_row(r, c2):
                    for j in range(D_MODEL // 16):
                        sl = pl.ds(j * 16, 16)
                        bufs[b][r, sl] = bufs[b][r, sl] * SCALE
                    return c2

                lax.fori_loop(0, G, scale_row, 0, unroll=4)
                store(g, b)

                gn = g + DEPTH
                bn = (b + DEPTH) % NBUF

                @pl.when(jnp.logical_and(gn >= NBUF, gn < gpw))
                def _():
                    wait_store(gn - NBUF, bn)

                @pl.when(gn < gpw)
                def _():
                    gather(gn, bn)
            return carry

        lax.fori_loop(0, gpw // NBUF, block, 0)

        for b in range(NBUF):
            wait_store(gpw - NBUF + b, b)

    return k(idx2d, table)


def kernel(x, emb_weight):
    b, s = x.shape
    idx2d = x.reshape(b * s // G, G).astype(jnp.int32)
    out = _gather_scale(idx2d, emb_weight)
    return out.reshape(b, s, D_MODEL)
